# baseline (speedup 1.0000x reference)
import functools

import jax
import jax.numpy as jnp
from jax import lax
from jax.experimental import pallas as pl
from jax.experimental.pallas import tpu as pltpu
from jax.experimental.pallas import tpu_sc as plsc

B = 4096
L = 200
D = 64
NC = 2
NS = 16
NW = NC * NS
RPW = B // NW


def _sc_embed_body(idx_hbm, table_hbm, out_hbm, idx_v, rows_v, out_v, sem):
    wid = lax.axis_index("s") * NC + lax.axis_index("c")
    base = wid * RPW

    def row_body(t, carry):
        r = base + t
        pltpu.sync_copy(idx_hbm.at[r], idx_v)
        cp1 = pltpu.async_copy(
            table_hbm.at[idx_v.at[pl.ds(0, 128)]], rows_v.at[pl.ds(0, 128)], sem)
        cp2 = pltpu.async_copy(
            table_hbm.at[idx_v.at[pl.ds(128, 72)]], rows_v.at[pl.ds(128, 72)], sem)
        cp1.wait()
        cp2.wait()

        def accum(i, acc):
            return tuple(acc[j] + rows_v[i, pl.ds(16 * j, 16)] for j in range(4))

        acc = lax.fori_loop(
            0, L, accum, tuple(jnp.zeros((16,), jnp.float32) for _ in range(4)))
        for j in range(4):
            out_v[pl.ds(16 * j, 16)] = acc[j] * (1.0 / L)
        pltpu.sync_copy(out_v, out_hbm.at[r])
        return carry

    lax.fori_loop(0, RPW, row_body, None)


_sc_embed = functools.partial(
    pl.kernel,
    out_type=jax.ShapeDtypeStruct((B, D), jnp.float32),
    mesh=plsc.VectorSubcoreMesh(
        core_axis_name="c", subcore_axis_name="s", num_cores=NC, num_subcores=NS),
    scratch_types=[
        pltpu.VMEM((L,), jnp.int32),
        pltpu.VMEM((L, D), jnp.float32),
        pltpu.VMEM((D,), jnp.float32),
        pltpu.SemaphoreType.DMA,
    ],
    compiler_params=pltpu.CompilerParams(use_tc_tiling_on_sc=False),
)(_sc_embed_body)


def _mlp_body(x_ref, w1_ref, b1_ref, w2_ref, b2_ref, o_ref):
    h = jnp.dot(x_ref[...], w1_ref[...], preferred_element_type=jnp.float32)
    h = jnp.maximum(h + b1_ref[...], 0.0)
    o_ref[...] = jnp.dot(h, w2_ref[...],
                         preferred_element_type=jnp.float32) + b2_ref[...]


def _mlp(x, W1, b1, W2, b2):
    return pl.pallas_call(
        _mlp_body,
        out_shape=jax.ShapeDtypeStruct((B, W2.shape[1]), jnp.float32),
    )(x, W1, b1, W2, b2)


def kernel(data_input, table, W1, b1, W2, b2):
    embedded = _sc_embed(data_input, table)
    return _mlp(embedded, W1, b1.reshape(1, -1), W2, b2.reshape(1, -1))
